# baseline (speedup 1.0000x reference)
import jax
import jax.numpy as jnp
from jax.experimental import pallas as pl
from jax.experimental.pallas import tpu as pltpu

B = 1024
M = 100000
KD = 16
VD = 16
G = M // 8
GP = 12544
MP = 8 * GP
M_BLK = 2048
NB = MP // M_BLK


def _prep_body(q_ref, kp_ref, vp_ref, qext_ref, kt_ref, vt_ref):
    kp = kp_ref[...]
    vp = vp_ref[...]
    n2 = jnp.sum(kp * kp, axis=1, keepdims=True)
    km2 = jnp.max(n2, axis=0, keepdims=True)
    q = q_ref[...]
    qn = jnp.sum(q * q, axis=1, keepdims=True)
    u = jnp.sqrt(qn * km2)
    qext_ref[:, 0:KD] = q.astype(jnp.bfloat16)
    qext_ref[:, KD : KD + 1] = (-u).astype(jnp.bfloat16)

    kt8 = kp.T.astype(jnp.bfloat16)
    vt8 = vp.T.astype(jnp.bfloat16)
    zpad = jnp.zeros((KD, GP - G), jnp.bfloat16)
    for r in range(8):
        kt_ref[0:KD, r * GP : r * GP + G] = kt8[KD * r : KD * r + KD, :]
        kt_ref[0:KD, r * GP + G : (r + 1) * GP] = zpad
        vt_ref[0:VD, r * GP : r * GP + G] = vt8[VD * r : VD * r + VD, :]
        vt_ref[0:VD, r * GP + G : (r + 1) * GP] = zpad
    col = jax.lax.broadcasted_iota(jnp.int32, (1, MP), 1)
    in_sec = col - (col // GP) * GP < G
    kt_ref[KD : KD + 1, :] = jnp.where(in_sec, 1.0, 64.0).astype(jnp.bfloat16)
    vt_ref[VD : VD + 1, :] = jnp.ones((1, MP), jnp.bfloat16)


def _main_body(qext_ref, kt_ref, vt_ref, o_ref, acc_ref):
    i = pl.program_id(0)

    @pl.when(i == 0)
    def _():
        acc_ref[...] = jnp.zeros_like(acc_ref)

    s = jnp.dot(qext_ref[...], kt_ref[...], preferred_element_type=jnp.float32)
    p = jnp.exp(s).astype(jnp.bfloat16)
    acc_ref[...] += jax.lax.dot_general(
        p, vt_ref[...], (((1,), (1,)), ((), ())),
        preferred_element_type=jnp.float32,
    )

    @pl.when(i == NB - 1)
    def _():
        o_ref[...] = acc_ref[:, 0:VD] / acc_ref[:, VD : VD + 1]


@jax.jit
def kernel(query, keys, values):
    kp = keys.reshape(G, 128)
    vp = values.reshape(G, 128)
    qext, kt_ext, vt_ext = pl.pallas_call(
        _prep_body,
        grid=(1,),
        in_specs=[
            pl.BlockSpec((B, KD), lambda t: (0, 0)),
            pl.BlockSpec((G, 128), lambda t: (0, 0)),
            pl.BlockSpec((G, 128), lambda t: (0, 0)),
        ],
        out_specs=[
            pl.BlockSpec((B, KD + 1), lambda t: (0, 0)),
            pl.BlockSpec((KD + 1, MP), lambda t: (0, 0)),
            pl.BlockSpec((VD + 1, MP), lambda t: (0, 0)),
        ],
        out_shape=[
            jax.ShapeDtypeStruct((B, KD + 1), jnp.bfloat16),
            jax.ShapeDtypeStruct((KD + 1, MP), jnp.bfloat16),
            jax.ShapeDtypeStruct((VD + 1, MP), jnp.bfloat16),
        ],
    )(query, kp, vp)
    return pl.pallas_call(
        _main_body,
        grid=(NB,),
        in_specs=[
            pl.BlockSpec((B, KD + 1), lambda i: (0, 0)),
            pl.BlockSpec((KD + 1, M_BLK), lambda i: (0, i)),
            pl.BlockSpec((VD + 1, M_BLK), lambda i: (0, i)),
        ],
        out_specs=pl.BlockSpec((B, VD), lambda i: (0, 0)),
        out_shape=jax.ShapeDtypeStruct((B, VD), jnp.float32),
        scratch_shapes=[
            pltpu.VMEM((B, VD + 1), jnp.float32),
        ],
    )(qext, kt_ext, vt_ext)


# speedup vs baseline: 1.1609x; 1.1609x over previous
import jax
import jax.numpy as jnp
from jax.experimental import pallas as pl
from jax.experimental.pallas import tpu as pltpu

B = 1024
M = 100000
KD = 16
VD = 16
M_BLK = 2000
NB = M // M_BLK
N0 = 10
M0_BLK = M // N0


def _qext_body(q_ref, ka_ref, qext_ref, km2_ref):
    t = pl.program_id(0)
    ka = ka_ref[...]
    n2 = jnp.sum(ka * ka, axis=1, keepdims=True)
    bmax = jnp.max(n2, axis=0, keepdims=True)

    @pl.when(t == 0)
    def _():
        km2_ref[...] = bmax

    @pl.when(t > 0)
    def _():
        km2_ref[...] = jnp.maximum(km2_ref[...], bmax)

    @pl.when(t == N0 - 1)
    def _():
        q = q_ref[...]
        qn = jnp.sum(q * q, axis=1, keepdims=True)
        u = jnp.sqrt(qn * km2_ref[...])
        qext_ref[:, 0:KD] = q.astype(jnp.bfloat16)
        qext_ref[:, KD : KD + 1] = (-u).astype(jnp.bfloat16)


def _main_body(qext_ref, k_ref, v_ref, o_ref, acc_ref):
    i = pl.program_id(0)

    @pl.when(i == 0)
    def _():
        acc_ref[...] = jnp.zeros_like(acc_ref)

    ones_col = jnp.ones((M_BLK, 1), jnp.bfloat16)
    k_ext = jnp.concatenate([k_ref[...].astype(jnp.bfloat16), ones_col], axis=1)
    s = jax.lax.dot_general(
        qext_ref[...], k_ext, (((1,), (1,)), ((), ())),
        preferred_element_type=jnp.float32,
    )
    p = jnp.exp(s).astype(jnp.bfloat16)
    v_ext = jnp.concatenate([v_ref[...].astype(jnp.bfloat16), ones_col], axis=1)
    acc_ref[...] += jnp.dot(p, v_ext, preferred_element_type=jnp.float32)

    @pl.when(i == NB - 1)
    def _():
        o_ref[...] = acc_ref[:, 0:VD] / acc_ref[:, VD : VD + 1]


@jax.jit
def kernel(query, keys, values):
    qext = pl.pallas_call(
        _qext_body,
        grid=(N0,),
        in_specs=[
            pl.BlockSpec((B, KD), lambda t: (0, 0)),
            pl.BlockSpec((M0_BLK, KD), lambda t: (t, 0)),
        ],
        out_specs=pl.BlockSpec((B, KD + 1), lambda t: (0, 0)),
        out_shape=jax.ShapeDtypeStruct((B, KD + 1), jnp.bfloat16),
        scratch_shapes=[pltpu.VMEM((1, 1), jnp.float32)],
    )(query, keys)
    return pl.pallas_call(
        _main_body,
        grid=(NB,),
        in_specs=[
            pl.BlockSpec((B, KD + 1), lambda i: (0, 0)),
            pl.BlockSpec((M_BLK, KD), lambda i: (i, 0)),
            pl.BlockSpec((M_BLK, VD), lambda i: (i, 0)),
        ],
        out_specs=pl.BlockSpec((B, VD), lambda i: (0, 0)),
        out_shape=jax.ShapeDtypeStruct((B, VD), jnp.float32),
        scratch_shapes=[pltpu.VMEM((B, VD + 1), jnp.float32)],
    )(qext, keys, values)


# speedup vs baseline: 1.2264x; 1.0565x over previous
import jax
import jax.numpy as jnp
from jax.experimental import pallas as pl
from jax.experimental.pallas import tpu as pltpu

B = 1024
M = 100000
KD = 16
VD = 16
M_BLK = 4096
MP = 102400
NB = MP // M_BLK


def _norms_body(ka_ref, o_ref):
    ka = ka_ref[...]
    n2 = jnp.sum(ka * ka, axis=1, keepdims=True)
    o_ref[...] = jnp.max(n2, axis=0, keepdims=True)


def _main_body(km2_ref, q_ref, kt_ref, vt_ref, o_ref, qext_ref, acc_ref):
    i = pl.program_id(0)

    @pl.when(i == 0)
    def _():
        q = q_ref[...]
        qn = jnp.sum(q * q, axis=1, keepdims=True)
        u = jnp.sqrt(qn * km2_ref[...])
        qext_ref[:, 0:KD] = q.astype(jnp.bfloat16)
        qext_ref[:, KD : KD + 1] = (-u).astype(jnp.bfloat16)
        acc_ref[...] = jnp.zeros_like(acc_ref)

    s = jnp.dot(qext_ref[...], kt_ref[...], preferred_element_type=jnp.float32)
    p = jnp.exp(s).astype(jnp.bfloat16)
    acc_ref[...] += jax.lax.dot_general(
        p, vt_ref[...], (((1,), (1,)), ((), ())),
        preferred_element_type=jnp.float32,
    )

    @pl.when(i == NB - 1)
    def _():
        o_ref[...] = acc_ref[:, 0:VD] / acc_ref[:, VD : VD + 1]


@jax.jit
def kernel(query, keys, values):
    kp = keys.reshape(M // 8, 128)
    km2 = pl.pallas_call(
        _norms_body,
        grid=(1,),
        in_specs=[pl.BlockSpec((M // 8, 128), lambda t: (0, 0))],
        out_specs=pl.BlockSpec((1, 1), lambda t: (0, 0)),
        out_shape=jax.ShapeDtypeStruct((1, 1), jnp.float32),
    )(kp)

    col = jax.lax.broadcasted_iota(jnp.int32, (1, MP), 1)
    bias_row = jnp.where(col < M, 1.0, 64.0).astype(jnp.bfloat16)
    kt_ext = jnp.concatenate(
        [jnp.pad(keys.T.astype(jnp.bfloat16), ((0, 0), (0, MP - M))), bias_row],
        axis=0,
    )
    vt_ext = jnp.concatenate(
        [
            jnp.pad(values.T.astype(jnp.bfloat16), ((0, 0), (0, MP - M))),
            jnp.ones((1, MP), jnp.bfloat16),
        ],
        axis=0,
    )
    return pl.pallas_call(
        _main_body,
        grid=(NB,),
        in_specs=[
            pl.BlockSpec((1, 1), lambda i: (0, 0)),
            pl.BlockSpec((B, KD), lambda i: (0, 0)),
            pl.BlockSpec((KD + 1, M_BLK), lambda i: (0, i)),
            pl.BlockSpec((VD + 1, M_BLK), lambda i: (0, i)),
        ],
        out_specs=pl.BlockSpec((B, VD), lambda i: (0, 0)),
        out_shape=jax.ShapeDtypeStruct((B, VD), jnp.float32),
        scratch_shapes=[
            pltpu.VMEM((B, KD + 1), jnp.bfloat16),
            pltpu.VMEM((B, VD + 1), jnp.float32),
        ],
    )(km2, query, kt_ext, vt_ext)


# speedup vs baseline: 1.5896x; 1.2961x over previous
import jax
import jax.numpy as jnp
from jax.experimental import pallas as pl
from jax.experimental.pallas import tpu as pltpu

B = 1024
M = 100000
KD = 16
VD = 16
M_BLK = 4096
MP = 102400
NB = MP // M_BLK


def _norms_body(kt_ref, o_ref, km2_ref):
    t = pl.program_id(0)
    ka = kt_ref[0:KD, :].astype(jnp.float32)
    n2 = jnp.sum(ka * ka, axis=0, keepdims=True)
    bmax = jnp.max(n2, axis=1, keepdims=True)

    @pl.when(t == 0)
    def _():
        km2_ref[...] = bmax

    @pl.when(t > 0)
    def _():
        km2_ref[...] = jnp.maximum(km2_ref[...], bmax)

    @pl.when(t == NB - 1)
    def _():
        o_ref[...] = km2_ref[...]


def _main_body(km2_ref, q_ref, kt_ref, vt_ref, o_ref, qext_ref, acc_ref):
    i = pl.program_id(0)

    @pl.when(i == 0)
    def _():
        q = q_ref[...]
        qn = jnp.sum(q * q, axis=1, keepdims=True)
        u = jnp.sqrt(qn * km2_ref[...])
        qext_ref[:, 0:KD] = q.astype(jnp.bfloat16)
        qext_ref[:, KD : KD + 1] = (-u).astype(jnp.bfloat16)
        acc_ref[...] = jnp.zeros_like(acc_ref)

    s = jnp.dot(qext_ref[...], kt_ref[...], preferred_element_type=jnp.float32)
    p = jnp.exp(s).astype(jnp.bfloat16)
    acc_ref[...] += jax.lax.dot_general(
        p, vt_ref[...], (((1,), (1,)), ((), ())),
        preferred_element_type=jnp.float32,
    )

    @pl.when(i == NB - 1)
    def _():
        o_ref[...] = acc_ref[:, 0:VD] / acc_ref[:, VD : VD + 1]


@jax.jit
def kernel(query, keys, values):
    bias_row = jnp.pad(
        jnp.ones((1, M), jnp.bfloat16), ((0, 0), (0, MP - M)),
        constant_values=64,
    )
    kt_ext = jnp.concatenate(
        [jnp.pad(keys.T.astype(jnp.bfloat16), ((0, 0), (0, MP - M))), bias_row],
        axis=0,
    )
    vt_ext = jnp.concatenate(
        [
            jnp.pad(values.T.astype(jnp.bfloat16), ((0, 0), (0, MP - M))),
            jnp.ones((1, MP), jnp.bfloat16),
        ],
        axis=0,
    )
    km2 = pl.pallas_call(
        _norms_body,
        grid=(NB,),
        in_specs=[pl.BlockSpec((KD + 1, M_BLK), lambda t: (0, t))],
        out_specs=pl.BlockSpec((1, 1), lambda t: (0, 0)),
        out_shape=jax.ShapeDtypeStruct((1, 1), jnp.float32),
        scratch_shapes=[pltpu.VMEM((1, 1), jnp.float32)],
    )(kt_ext)
    return pl.pallas_call(
        _main_body,
        grid=(NB,),
        in_specs=[
            pl.BlockSpec((1, 1), lambda i: (0, 0)),
            pl.BlockSpec((B, KD), lambda i: (0, 0)),
            pl.BlockSpec((KD + 1, M_BLK), lambda i: (0, i)),
            pl.BlockSpec((VD + 1, M_BLK), lambda i: (0, i)),
        ],
        out_specs=pl.BlockSpec((B, VD), lambda i: (0, 0)),
        out_shape=jax.ShapeDtypeStruct((B, VD), jnp.float32),
        scratch_shapes=[
            pltpu.VMEM((B, KD + 1), jnp.bfloat16),
            pltpu.VMEM((B, VD + 1), jnp.float32),
        ],
    )(km2, query, kt_ext, vt_ext)


# speedup vs baseline: 1.7116x; 1.0767x over previous
import jax
import jax.numpy as jnp
from jax.experimental import pallas as pl
from jax.experimental.pallas import tpu as pltpu

B = 1024
M = 100000
KD = 16
VD = 16
M_BLK = 4096
MP = 102400
NB = MP // M_BLK


def _norms_body(kt_ref, o_ref):
    ka = kt_ref[0:KD, :].astype(jnp.float32)
    n2 = jnp.sum(ka * ka, axis=0, keepdims=True)
    o_ref[...] = jnp.max(n2, axis=1, keepdims=True)


def _main_body(km2_ref, q_ref, kt_ref, vt_ref, o_ref, qext_ref, acc_ref):
    i = pl.program_id(0)

    @pl.when(i == 0)
    def _():
        q = q_ref[...]
        qn = jnp.sum(q * q, axis=1, keepdims=True)
        u = jnp.sqrt(qn * km2_ref[...])
        qext_ref[:, 0:KD] = q.astype(jnp.bfloat16)
        qext_ref[:, KD : KD + 1] = (-u).astype(jnp.bfloat16)
        acc_ref[...] = jnp.zeros_like(acc_ref)

    s = jnp.dot(qext_ref[...], kt_ref[...], preferred_element_type=jnp.float32)
    p = jnp.exp(s).astype(jnp.bfloat16)
    acc_ref[...] += jax.lax.dot_general(
        p, vt_ref[...], (((1,), (1,)), ((), ())),
        preferred_element_type=jnp.float32,
    )

    @pl.when(i == NB - 1)
    def _():
        o_ref[...] = acc_ref[:, 0:VD] / acc_ref[:, VD : VD + 1]


@jax.jit
def kernel(query, keys, values):
    bias_row = jnp.pad(
        jnp.ones((1, M), jnp.bfloat16), ((0, 0), (0, MP - M)),
        constant_values=64,
    )
    kt_ext = jnp.concatenate(
        [jnp.pad(keys.T.astype(jnp.bfloat16), ((0, 0), (0, MP - M))), bias_row],
        axis=0,
    )
    vt_ext = jnp.concatenate(
        [
            jnp.pad(values.T.astype(jnp.bfloat16), ((0, 0), (0, MP - M))),
            jnp.ones((1, MP), jnp.bfloat16),
        ],
        axis=0,
    )
    km2 = pl.pallas_call(
        _norms_body,
        grid=(1,),
        in_specs=[pl.BlockSpec((KD + 1, MP), lambda t: (0, 0))],
        out_specs=pl.BlockSpec((1, 1), lambda t: (0, 0)),
        out_shape=jax.ShapeDtypeStruct((1, 1), jnp.float32),
    )(kt_ext)
    return pl.pallas_call(
        _main_body,
        grid=(NB,),
        in_specs=[
            pl.BlockSpec((1, 1), lambda i: (0, 0)),
            pl.BlockSpec((B, KD), lambda i: (0, 0)),
            pl.BlockSpec((KD + 1, M_BLK), lambda i: (0, i)),
            pl.BlockSpec((VD + 1, M_BLK), lambda i: (0, i)),
        ],
        out_specs=pl.BlockSpec((B, VD), lambda i: (0, 0)),
        out_shape=jax.ShapeDtypeStruct((B, VD), jnp.float32),
        scratch_shapes=[
            pltpu.VMEM((B, KD + 1), jnp.bfloat16),
            pltpu.VMEM((B, VD + 1), jnp.float32),
        ],
    )(km2, query, kt_ext, vt_ext)
